# baseline (speedup 1.0000x reference)
import functools

import jax
import jax.numpy as jnp
from jax import lax
from jax.experimental import pallas as pl
from jax.experimental.pallas import tpu as pltpu
from jax.experimental.pallas import tpu_sc as plsc

ALPHA = 200.0
B, N, M = 16, 512, 512
LANES = 16
CHUNK = 16
NCH = N // CHUNK
CVECS = M // LANES
UNROLL = 8
GCOLS = UNROLL * LANES


def _sm_body(s_hbm, nrow_hbm, ncol_hbm, out_hbm, buf, buf_t, zbuf,
             nrow_v, ncol_v, sem):
    wid = lax.axis_index("s") * 2 + lax.axis_index("c")

    pltpu.sync_copy(nrow_hbm, nrow_v)
    pltpu.sync_copy(ncol_hbm, ncol_v)

    lanes = lax.iota(jnp.int32, LANES)
    zvec = jnp.zeros((LANES,), jnp.float32)
    idx0 = lanes * M

    def _zinit(j, carry):
        zbuf[j // CVECS, pl.ds((j % CVECS) * LANES, LANES)] = zvec
        return carry

    lax.fori_loop(0, CHUNK * CVECS, _zinit, 0)

    nv = nrow_v[...]
    mv = ncol_v[...]

    def _batch(b, carry0):
        bf = jnp.full((LANES,), b, jnp.int32)
        n = nv.at[bf].get(mode="promise_in_bounds")[0]
        m = mv.at[bf].get(mode="promise_in_bounds")[0]
        ch = lax.rem(wid + 2 * b, NCH)
        r0 = ch * CHUNK
        nblk = (m + LANES - 1) // LANES
        m8 = (m // UNROLL) * UNROLL

        @pl.when(r0 < n)
        def _compute():
            copies = [
                pltpu.async_copy(
                    s_hbm.at[b, r0 + r, :], buf.at[pl.ds(r * M, M)], sem)
                for r in range(CHUNK)
            ]
            for h in copies:
                h.wait()
            rowv = (r0 + lanes) < n

            def _p1(g, carry):
                accs, idx = carry
                off = g * GCOLS
                new = []
                for u in range(UNROLL):
                    x = plsc.load_gather(buf, [idx + u])
                    buf_t[pl.ds(off + u * LANES, LANES)] = x
                    new.append(jnp.maximum(accs[u], x))
                return tuple(new), idx + UNROLL

            def _p1_col(c, acc):
                x = plsc.load_gather(buf, [idx0 + c])
                buf_t[pl.ds(c * LANES, LANES)] = x
                return jnp.maximum(acc, x)

            acc0 = jnp.full((LANES,), -3.0e38, jnp.float32)
            mvecs, _ = lax.fori_loop(
                0, m8 // UNROLL, _p1, ((acc0,) * UNROLL, idx0))
            mvec = functools.reduce(jnp.maximum, mvecs)
            rowmax = lax.fori_loop(m8, m, _p1_col, mvec)

            def _p2(g, accs):
                off = g * GCOLS
                new = []
                for u in range(UNROLL):
                    x = buf_t[pl.ds(off + u * LANES, LANES)]
                    e = jnp.exp((x - rowmax) * ALPHA)
                    buf_t[pl.ds(off + u * LANES, LANES)] = e
                    new.append(accs[u] + e)
                return tuple(new)

            def _p2_col(c, acc):
                x = buf_t[pl.ds(c * LANES, LANES)]
                e = jnp.exp((x - rowmax) * ALPHA)
                buf_t[pl.ds(c * LANES, LANES)] = e
                return acc + e

            svecs = lax.fori_loop(0, m8 // UNROLL, _p2, (zvec,) * UNROLL)
            svec = functools.reduce(jnp.add, svecs)
            denom = lax.fori_loop(m8, m, _p2_col, svec)
            scale = jnp.where(rowv, 1.0 / denom, 0.0)

            def _p3(g, idx):
                off = g * GCOLS
                for u in range(UNROLL):
                    e = buf_t[pl.ds(off + u * LANES, LANES)]
                    plsc.store_scatter(buf, [idx + u], e * scale)
                return idx + UNROLL

            def _p3_col(c, carry):
                e = buf_t[pl.ds(c * LANES, LANES)]
                plsc.store_scatter(buf, [idx0 + c], e * scale)
                return carry

            lax.fori_loop(0, m8 // UNROLL, _p3, idx0)
            lax.fori_loop(m8, m, _p3_col, 0)

            def _ztcol(c, carry):
                plsc.store_scatter(buf, [idx0 + c], zvec)
                return carry

            lax.fori_loop(m, nblk * LANES, _ztcol, 0)

            def _ztrow(r, carry):
                def _ztblk(cb, carry2):
                    buf[pl.ds(r * M + cb * LANES, LANES)] = zvec
                    return carry2

                lax.fori_loop(nblk, CVECS, _ztblk, 0)
                return carry

            lax.fori_loop(0, CHUNK, _ztrow, 0)

            out_copies = [
                pltpu.async_copy(
                    buf.at[pl.ds(r * M, M)], out_hbm.at[b, r0 + r, :], sem)
                for r in range(CHUNK)
            ]
            for h in out_copies:
                h.wait()

        @pl.when(r0 >= n)
        def _zero():
            pltpu.sync_copy(zbuf, out_hbm.at[b, pl.ds(r0, CHUNK), :])

        return carry0

    lax.fori_loop(0, B, _batch, 0)


@jax.jit
def _sm_call(s, nrow_gt, ncol_gt):
    mesh = plsc.VectorSubcoreMesh(core_axis_name="c", subcore_axis_name="s")
    return pl.kernel(
        _sm_body,
        mesh=mesh,
        compiler_params=pltpu.CompilerParams(needs_layout_passes=False),
        out_type=jax.ShapeDtypeStruct((B, N, M), jnp.float32),
        scratch_types=[
            pltpu.VMEM((CHUNK * M,), jnp.float32),
            pltpu.VMEM((M * CHUNK,), jnp.float32),
            pltpu.VMEM((CHUNK, M), jnp.float32),
            pltpu.VMEM((LANES,), jnp.int32),
            pltpu.VMEM((LANES,), jnp.int32),
            pltpu.SemaphoreType.DMA,
        ],
    )(s, nrow_gt, ncol_gt)


def kernel(s, nrow_gt, ncol_gt):
    return _sm_call(s, nrow_gt, ncol_gt)
